# Initial kernel scaffold; baseline (speedup 1.0000x reference)
import jax
import jax.numpy as jnp
from jax.experimental import pallas as pl


def kernel(x, edge_index, W, b):
    raise NotImplementedError("write your pallas kernel here")


# speedup vs baseline: 4.8597x; 4.8597x over previous
import functools

import jax
import jax.numpy as jnp
from jax import lax
from jax.experimental import pallas as pl
from jax.experimental.pallas import tpu as pltpu
from jax.experimental.pallas import tpu_sc as plsc

N_NODES = 10000
N_EDGES = 320000
D = 128
DA = 144
NC, NS = 2, 16
NW = NC * NS
C = 128
EPW = 10112
E_PAD = EPW * NW
CHUNKS = EPW // C
N_PAD = 10112
RPT = N_PAD // NS


def _sc_segment_accumulate(x_aug, src_p, dst_p):
  mesh = plsc.VectorSubcoreMesh(core_axis_name="c", subcore_axis_name="s")

  @functools.partial(
      pl.kernel,
      out_type=jax.ShapeDtypeStruct((NC, N_PAD, DA), jnp.float32),
      mesh=mesh,
      compiler_params=pltpu.CompilerParams(use_tc_tiling_on_sc=False),
      scratch_types=[
          pltpu.VMEM((C,), jnp.int32),
          pltpu.VMEM((C,), jnp.int32),
          pltpu.VMEM((C, DA), jnp.float32),
          pltpu.VMEM_SHARED((N_PAD, DA), jnp.float32),
          pltpu.SemaphoreType.DMA,
      ],
  )
  def sc_fn(x_hbm, src_hbm, dst_hbm, out_hbm, src_v, dst_v, rows_v, agg_sh,
            sem):
    cid = lax.axis_index("c")
    sid = lax.axis_index("s")
    wid = sid * NC + cid

    zeros16 = jnp.zeros((16,), jnp.float32)

    def zero_row(i, _):
      for j in range(DA // 16):
        rows_v[i, j * 16:(j + 1) * 16] = zeros16
      return 0

    lax.fori_loop(0, C, zero_row, 0)
    tr0 = sid * RPT
    for k in range(RPT // C):
      pltpu.sync_copy(rows_v, agg_sh.at[pl.ds(tr0 + k * C, C)])
    rem = RPT % C
    if rem:
      pltpu.sync_copy(rows_v.at[pl.ds(0, rem)],
                      agg_sh.at[pl.ds(tr0 + (RPT // C) * C, rem)])
    plsc.subcore_barrier()

    ebase = wid * EPW

    def chunk_body(g, _):
      base = ebase + g * C
      pltpu.sync_copy(src_hbm.at[pl.ds(base, C)], src_v)
      pltpu.sync_copy(dst_hbm.at[pl.ds(base, C)], dst_v)
      pltpu.async_copy(x_hbm.at[src_v], rows_v, sem).wait()
      pltpu.sync_copy(rows_v, agg_sh.at[dst_v], add=True)
      return 0

    lax.fori_loop(0, CHUNKS, chunk_body, 0)
    plsc.subcore_barrier()

    pltpu.sync_copy(agg_sh.at[pl.ds(tr0, RPT)],
                    out_hbm.at[cid].at[pl.ds(tr0, RPT)])

  return sc_fn(x_aug, src_p, dst_p)


def _tc_finish(partials, x, W, b2):
  BR = 400
  grid = (N_NODES // BR,)

  def tc_fn(p_ref, x_ref, w_ref, b_ref, o_ref):
    p = p_ref[...]
    s = p[0] + p[1]
    agg = s[:, :D]
    deg = jnp.maximum(s[:, D:D + 1], 1.0)
    h = jnp.dot(agg / deg + x_ref[...], w_ref[...],
                preferred_element_type=jnp.float32)
    o_ref[...] = jnp.maximum(h + b_ref[...], 0.0)

  return pl.pallas_call(
      tc_fn,
      grid=grid,
      in_specs=[
          pl.BlockSpec((NC, BR, DA), lambda i: (0, i, 0)),
          pl.BlockSpec((BR, D), lambda i: (i, 0)),
          pl.BlockSpec((D, D), lambda i: (0, 0)),
          pl.BlockSpec((1, D), lambda i: (0, 0)),
      ],
      out_specs=pl.BlockSpec((BR, D), lambda i: (i, 0)),
      out_shape=jax.ShapeDtypeStruct((N_NODES, D), jnp.float32),
  )(partials, x, W, b2)


@jax.jit
def kernel(x, edge_index, W, b):
  src = edge_index[0]
  dst = edge_index[1]
  pad = E_PAD - N_EDGES
  src_p = jnp.concatenate([src, jnp.zeros((pad,), jnp.int32)])
  dst_p = jnp.concatenate([dst, jnp.full((pad,), N_NODES, jnp.int32)])
  ones_col = jnp.ones((N_NODES, 1), jnp.float32)
  zpad = jnp.zeros((N_NODES, DA - D - 1), jnp.float32)
  x_aug = jnp.concatenate([x, ones_col, zpad], axis=1)

  partials = _sc_segment_accumulate(x_aug, src_p, dst_p)
  return _tc_finish(partials, x, W, b.reshape(1, D))
